# Initial kernel scaffold; baseline (speedup 1.0000x reference)
import functools

import jax
import jax.numpy as jnp
from jax import lax
from jax.experimental import pallas as pl
from jax.experimental.pallas import tpu as pltpu
from jax.experimental.pallas import tpu_sc as plsc

N_NODES = 10000
D = 128
CHUNK = 128
N_TILES = 16
N_CORES = 2
NW = N_CORES * N_TILES
N_PAD = 10240
SLICE = N_PAD // N_TILES


def _make_counts_kernel(n_chunks: int):
    base_chunks = n_chunks // NW
    extra = n_chunks - base_chunks * NW
    mesh = plsc.VectorSubcoreMesh(core_axis_name="c", subcore_axis_name="s")

    @functools.partial(
        pl.kernel,
        mesh=mesh,
        out_type=jax.ShapeDtypeStruct((N_CORES, N_PAD), jnp.float32),
        scratch_types=[
            pltpu.VMEM((base_chunks + 1, CHUNK), jnp.int32),
            pltpu.VMEM((CHUNK,), jnp.float32),
            pltpu.VMEM((SLICE,), jnp.float32),
            pltpu.VMEM_SHARED((N_PAD,), jnp.float32),
        ],
    )
    def counts_kernel(idx_hbm, out_hbm, idx_v, ones_v, zero_v, counts_sh):
        cid = lax.axis_index("c")
        sid = lax.axis_index("s")
        wid = cid * N_TILES + sid

        for i in range(CHUNK // 16):
            ones_v[pl.ds(i * 16, 16)] = jnp.full((16,), 1.0, jnp.float32)
        for i in range(SLICE // 16):
            zero_v[pl.ds(i * 16, 16)] = jnp.zeros((16,), jnp.float32)

        pltpu.sync_copy(zero_v, counts_sh.at[pl.ds(sid * SLICE, SLICE)])
        plsc.subcore_barrier()

        row0 = wid * base_chunks
        pltpu.sync_copy(idx_hbm.at[pl.ds(row0, base_chunks)],
                        idx_v.at[pl.ds(0, base_chunks)])

        @pl.when(wid < extra)
        def _():
            pltpu.sync_copy(idx_hbm.at[pl.ds(NW * base_chunks + wid, 1)],
                            idx_v.at[pl.ds(base_chunks, 1)])

        def body(j, carry):
            pltpu.sync_copy(ones_v, counts_sh.at[idx_v.at[j]], add=True)
            return carry

        lax.fori_loop(0, base_chunks, body, 0)

        @pl.when(wid < extra)
        def _():
            pltpu.sync_copy(ones_v, counts_sh.at[idx_v.at[base_chunks]],
                            add=True)

        plsc.subcore_barrier()
        pltpu.sync_copy(counts_sh.at[pl.ds(sid * SLICE, SLICE)],
                        out_hbm.at[cid, pl.ds(sid * SLICE, SLICE)])

    return counts_kernel


def _mask_mul_kernel(counts_ref, v_ref, out_ref):
    total = counts_ref[0] + counts_ref[1]
    mask = total[:N_NODES] > 0.0
    out_ref[...] = jnp.where(mask, v_ref[...], 0.0)


def kernel(q, k, v, self_indices, neighbor_indices):
    del q, k, neighbor_indices
    e = self_indices.shape[0]
    idx2d = self_indices.astype(jnp.int32).reshape(e // CHUNK, CHUNK)
    counts = _make_counts_kernel(e // CHUNK)(idx2d)
    out = pl.pallas_call(
        _mask_mul_kernel,
        out_shape=jax.ShapeDtypeStruct((N_NODES, D), jnp.float32),
    )(counts.reshape(N_CORES, N_PAD, 1), v)
    return out


# speedup vs baseline: 114.8803x; 114.8803x over previous
import functools

import jax
import jax.numpy as jnp
from jax import lax
from jax.experimental import pallas as pl
from jax.experimental.pallas import tpu as pltpu
from jax.experimental.pallas import tpu_sc as plsc

N_NODES = 10000
D = 128
CHUNK = 128
N_TILES = 16
N_CORES = 2
NW = N_CORES * N_TILES
N_PAD = 10240
SLICE = N_PAD // N_TILES


def _make_counts_kernel(rows: int):
    assert rows % (NW * 8) == 0
    base_rows = rows // NW
    mesh = plsc.VectorSubcoreMesh(core_axis_name="c", subcore_axis_name="s")

    @functools.partial(
        pl.kernel,
        mesh=mesh,
        out_type=jax.ShapeDtypeStruct((N_CORES * N_PAD,), jnp.float32),
        scratch_types=[
            pltpu.VMEM((base_rows, CHUNK), jnp.int32),
            pltpu.VMEM((CHUNK,), jnp.float32),
            pltpu.VMEM((SLICE,), jnp.float32),
            pltpu.VMEM_SHARED((N_PAD,), jnp.float32),
        ],
    )
    def counts_kernel(idx_hbm, out_hbm, idx_v, ones_v, zero_v, counts_sh):
        cid = lax.axis_index("c")
        sid = lax.axis_index("s")
        wid = cid * N_TILES + sid

        for i in range(CHUNK // 16):
            ones_v[pl.ds(i * 16, 16)] = jnp.full((16,), 1.0, jnp.float32)
        for i in range(SLICE // 16):
            zero_v[pl.ds(i * 16, 16)] = jnp.zeros((16,), jnp.float32)

        pltpu.sync_copy(zero_v, counts_sh.at[pl.ds(sid * SLICE, SLICE)])
        plsc.subcore_barrier()

        pltpu.sync_copy(idx_hbm.at[pl.ds(wid * base_rows, base_rows)], idx_v)

        def body(j, carry):
            pltpu.sync_copy(ones_v, counts_sh.at[idx_v.at[j]], add=True)
            return carry

        lax.fori_loop(0, base_rows, body, 0)

        plsc.subcore_barrier()
        pltpu.sync_copy(
            counts_sh.at[pl.ds(sid * SLICE, SLICE)],
            out_hbm.at[pl.ds(cid * N_PAD + sid * SLICE, SLICE)])

    return counts_kernel


def _mask_mul_kernel(counts_ref, v_ref, out_ref):
    total = counts_ref[0] + counts_ref[1]
    mask = total[:N_NODES] > 0.0
    out_ref[...] = jnp.where(mask, v_ref[...], 0.0)


def kernel(q, k, v, self_indices, neighbor_indices):
    del q, k, neighbor_indices
    e = self_indices.shape[0]
    rows = -(-e // (CHUNK * NW * 8)) * (NW * 8)
    idx_flat = self_indices.astype(jnp.int32)
    pad = jnp.full((rows * CHUNK - e,), N_NODES, jnp.int32)
    idx2d = jnp.concatenate([idx_flat, pad]).reshape(rows, CHUNK)
    counts = _make_counts_kernel(rows)(idx2d)
    out = pl.pallas_call(
        _mask_mul_kernel,
        out_shape=jax.ShapeDtypeStruct((N_NODES, D), jnp.float32),
    )(counts.reshape(N_CORES, N_PAD, 1), v)
    return out


# speedup vs baseline: 152.3996x; 1.3266x over previous
import functools

import jax
import jax.numpy as jnp
from jax import lax
from jax.experimental import pallas as pl
from jax.experimental.pallas import tpu as pltpu
from jax.experimental.pallas import tpu_sc as plsc

N_NODES = 10000
D = 128
CHUNK = 128
N_TILES = 16
N_CORES = 2
NW = N_CORES * N_TILES
N_PAD = 10240
SLICE = N_PAD // N_TILES


def _make_counts_kernel(rows: int):
    assert rows % (NW * 8) == 0
    base_rows = rows // NW
    mesh = plsc.VectorSubcoreMesh(core_axis_name="c", subcore_axis_name="s")

    group = 16
    assert base_rows % group == 0

    @functools.partial(
        pl.kernel,
        mesh=mesh,
        out_type=jax.ShapeDtypeStruct((N_CORES * N_PAD,), jnp.float32),
        scratch_types=[
            pltpu.VMEM((base_rows, CHUNK), jnp.int32),
            pltpu.VMEM((CHUNK,), jnp.float32),
            pltpu.VMEM((SLICE,), jnp.float32),
            pltpu.VMEM_SHARED((N_PAD,), jnp.float32),
            pltpu.SemaphoreType.DMA,
            pltpu.SemaphoreType.DMA,
        ],
    )
    def counts_kernel(idx_hbm, out_hbm, idx_v, ones_v, zero_v, counts_sh,
                      stage_sem, scat_sem):
        cid = lax.axis_index("c")
        sid = lax.axis_index("s")
        wid = cid * N_TILES + sid

        stage = pltpu.async_copy(
            idx_hbm.at[pl.ds(wid * base_rows, base_rows)], idx_v, stage_sem)

        for i in range(CHUNK // 16):
            ones_v[pl.ds(i * 16, 16)] = jnp.full((16,), 1.0, jnp.float32)
        for i in range(SLICE // 16):
            zero_v[pl.ds(i * 16, 16)] = jnp.zeros((16,), jnp.float32)

        pltpu.sync_copy(zero_v, counts_sh.at[pl.ds(sid * SLICE, SLICE)])
        stage.wait()
        plsc.subcore_barrier()

        def body(g, carry):
            descs = [
                pltpu.async_copy(ones_v,
                                 counts_sh.at[idx_v.at[g * group + j]],
                                 scat_sem, add=True)
                for j in range(group)
            ]
            for d in descs:
                d.wait()
            return carry

        lax.fori_loop(0, base_rows // group, body, 0)

        plsc.subcore_barrier()
        pltpu.sync_copy(
            counts_sh.at[pl.ds(sid * SLICE, SLICE)],
            out_hbm.at[pl.ds(cid * N_PAD + sid * SLICE, SLICE)])

    return counts_kernel


def _mask_mul_kernel(counts_ref, v_ref, out_ref):
    total = counts_ref[pl.ds(0, N_PAD)] + counts_ref[pl.ds(N_PAD, N_PAD)]
    m = (total > 0.0).astype(jnp.float32).reshape(1, N_PAD)
    m_col = jnp.transpose(m)
    out_ref[...] = v_ref[...] * m_col[:N_NODES]


def kernel(q, k, v, self_indices, neighbor_indices):
    del q, k, neighbor_indices
    e = self_indices.shape[0]
    rows = -(-e // (CHUNK * NW * 8)) * (NW * 8)
    idx_flat = self_indices.astype(jnp.int32)
    pad = jnp.full((rows * CHUNK - e,), N_NODES, jnp.int32)
    idx2d = jnp.concatenate([idx_flat, pad]).reshape(rows, CHUNK)
    counts = _make_counts_kernel(rows)(idx2d)
    out = pl.pallas_call(
        _mask_mul_kernel,
        out_shape=jax.ShapeDtypeStruct((N_NODES, D), jnp.float32),
    )(counts, v)
    return out


# speedup vs baseline: 171.5330x; 1.1255x over previous
import functools

import jax
import jax.numpy as jnp
from jax import lax
from jax.experimental import pallas as pl
from jax.experimental.pallas import tpu as pltpu
from jax.experimental.pallas import tpu_sc as plsc

N_NODES = 10000
D = 128
CHUNK = 128
N_TILES = 16
N_CORES = 2
NW = N_CORES * N_TILES
N_PAD = 10240
SLICE = N_PAD // N_TILES


def _make_counts_kernel(rows: int):
    assert rows % (NW * 8) == 0
    base_rows = rows // NW
    mesh = plsc.VectorSubcoreMesh(core_axis_name="c", subcore_axis_name="s")

    group = 16
    assert base_rows % group == 0

    @functools.partial(
        pl.kernel,
        mesh=mesh,
        out_type=[jax.ShapeDtypeStruct((N_PAD,), jnp.float32),
                  jax.ShapeDtypeStruct((N_PAD,), jnp.float32)],
        scratch_types=[
            pltpu.VMEM((base_rows, CHUNK), jnp.int32),
            pltpu.VMEM((CHUNK,), jnp.float32),
            pltpu.VMEM((SLICE,), jnp.float32),
            pltpu.VMEM_SHARED((N_PAD,), jnp.float32),
            pltpu.SemaphoreType.DMA,
            pltpu.SemaphoreType.DMA,
        ],
    )
    def counts_kernel(idx_hbm, out0_hbm, out1_hbm, idx_v, ones_v, zero_v,
                      counts_sh, stage_sem, scat_sem):
        cid = lax.axis_index("c")
        sid = lax.axis_index("s")
        wid = cid * N_TILES + sid

        stage = pltpu.async_copy(
            idx_hbm.at[pl.ds(wid * base_rows, base_rows)], idx_v, stage_sem)

        for i in range(CHUNK // 16):
            ones_v[pl.ds(i * 16, 16)] = jnp.full((16,), 1.0, jnp.float32)
        for i in range(SLICE // 16):
            zero_v[pl.ds(i * 16, 16)] = jnp.zeros((16,), jnp.float32)

        pltpu.sync_copy(zero_v, counts_sh.at[pl.ds(sid * SLICE, SLICE)])
        stage.wait()
        plsc.subcore_barrier()

        def body(g, carry):
            descs = [
                pltpu.async_copy(ones_v,
                                 counts_sh.at[idx_v.at[g * group + j]],
                                 scat_sem, add=True)
                for j in range(group)
            ]
            for d in descs:
                d.wait()
            return carry

        lax.fori_loop(0, base_rows // group, body, 0)

        plsc.subcore_barrier()
        @pl.when(cid == 0)
        def _():
            pltpu.sync_copy(counts_sh.at[pl.ds(sid * SLICE, SLICE)],
                            out0_hbm.at[pl.ds(sid * SLICE, SLICE)])

        @pl.when(cid == 1)
        def _():
            pltpu.sync_copy(counts_sh.at[pl.ds(sid * SLICE, SLICE)],
                            out1_hbm.at[pl.ds(sid * SLICE, SLICE)])

    return counts_kernel


ROWS_BLK = 1024


def _mask_mul_kernel(c0_ref, c1_ref, v_ref, out_ref):
    total = c0_ref[...] + c1_ref[...]
    m = (total > 0.0).astype(jnp.float32).reshape(1, ROWS_BLK)
    out_ref[...] = v_ref[...] * jnp.transpose(m)


def kernel(q, k, v, self_indices, neighbor_indices):
    del q, k, neighbor_indices
    e = self_indices.shape[0]
    rows = -(-e // (CHUNK * NW * 8)) * (NW * 8)
    n_pad_idx = rows * CHUNK - e
    idx_flat = self_indices.astype(jnp.int32)
    pad = N_NODES + (jnp.arange(n_pad_idx, dtype=jnp.int32)
                     % (N_PAD - N_NODES))
    idx2d = jnp.concatenate([idx_flat, pad]).reshape(rows, CHUNK)
    c0, c1 = _make_counts_kernel(rows)(idx2d)
    out = pl.pallas_call(
        _mask_mul_kernel,
        grid=(-(-N_NODES // ROWS_BLK),),
        in_specs=[
            pl.BlockSpec((ROWS_BLK,), lambda i: (i,)),
            pl.BlockSpec((ROWS_BLK,), lambda i: (i,)),
            pl.BlockSpec((ROWS_BLK, D), lambda i: (i, 0)),
        ],
        out_specs=pl.BlockSpec((ROWS_BLK, D), lambda i: (i, 0)),
        out_shape=jax.ShapeDtypeStruct((N_NODES, D), jnp.float32),
    )(c0, c1, v)
    return out


# speedup vs baseline: 186.3470x; 1.0864x over previous
import functools

import jax
import jax.numpy as jnp
from jax import lax
from jax.experimental import pallas as pl
from jax.experimental.pallas import tpu as pltpu
from jax.experimental.pallas import tpu_sc as plsc

N_NODES = 10000
D = 128
CHUNK = 128
N_TILES = 16
N_CORES = 2
NW = N_CORES * N_TILES
N_PAD = 10240
SLICE = N_PAD // N_TILES


def _make_counts_kernel(rows: int):
    assert rows % (NW * 8) == 0
    base_rows = rows // NW
    mesh = plsc.VectorSubcoreMesh(core_axis_name="c", subcore_axis_name="s")

    group = 16
    assert base_rows % group == 0

    @functools.partial(
        pl.kernel,
        mesh=mesh,
        out_type=[jax.ShapeDtypeStruct((N_PAD,), jnp.float32),
                  jax.ShapeDtypeStruct((N_PAD,), jnp.float32)],
        scratch_types=[
            pltpu.VMEM((base_rows, CHUNK), jnp.int32),
            pltpu.VMEM((CHUNK,), jnp.float32),
            pltpu.VMEM((SLICE,), jnp.float32),
            pltpu.VMEM_SHARED((N_PAD,), jnp.float32),
            pltpu.SemaphoreType.DMA,
            pltpu.SemaphoreType.DMA,
        ],
    )
    def counts_kernel(idx_hbm, out0_hbm, out1_hbm, idx_v, ones_v, zero_v,
                      counts_sh, stage_sem, scat_sem):
        cid = lax.axis_index("c")
        sid = lax.axis_index("s")
        wid = cid * N_TILES + sid

        stage = pltpu.async_copy(
            idx_hbm.at[pl.ds(wid * base_rows, base_rows)], idx_v, stage_sem)

        for i in range(CHUNK // 16):
            ones_v[pl.ds(i * 16, 16)] = jnp.full((16,), 1.0, jnp.float32)
        for i in range(SLICE // 16):
            zero_v[pl.ds(i * 16, 16)] = jnp.zeros((16,), jnp.float32)

        pltpu.sync_copy(zero_v, counts_sh.at[pl.ds(sid * SLICE, SLICE)])
        stage.wait()
        plsc.subcore_barrier()

        def body(g, carry):
            descs = [
                pltpu.async_copy(ones_v,
                                 counts_sh.at[idx_v.at[g * group + j]],
                                 scat_sem, add=True)
                for j in range(group)
            ]
            for d in descs:
                d.wait()
            return carry

        lax.fori_loop(0, base_rows // group, body, 0)

        plsc.subcore_barrier()
        @pl.when(cid == 0)
        def _():
            pltpu.sync_copy(counts_sh.at[pl.ds(sid * SLICE, SLICE)],
                            out0_hbm.at[pl.ds(sid * SLICE, SLICE)])

        @pl.when(cid == 1)
        def _():
            pltpu.sync_copy(counts_sh.at[pl.ds(sid * SLICE, SLICE)],
                            out1_hbm.at[pl.ds(sid * SLICE, SLICE)])

    return counts_kernel


ROWS_BLK = 2048


def _mask_mul_kernel(c0_ref, c1_ref, v_ref, out_ref):
    total = c0_ref[...] + c1_ref[...]
    m = (total > 0.0).astype(jnp.float32)
    m2 = jax.lax.broadcast_in_dim(m, (ROWS_BLK, D), (0,))
    out_ref[...] = v_ref[...] * m2


def kernel(q, k, v, self_indices, neighbor_indices):
    del q, k, neighbor_indices
    e = self_indices.shape[0]
    rows = -(-e // (CHUNK * NW * 8)) * (NW * 8)
    n_pad_idx = rows * CHUNK - e
    idx_flat = self_indices.astype(jnp.int32)
    pad = N_NODES + (jnp.arange(n_pad_idx, dtype=jnp.int32)
                     % (N_PAD - N_NODES))
    idx2d = jnp.concatenate([idx_flat, pad]).reshape(rows, CHUNK)
    c0, c1 = _make_counts_kernel(rows)(idx2d)
    out = pl.pallas_call(
        _mask_mul_kernel,
        grid=(-(-N_NODES // ROWS_BLK),),
        in_specs=[
            pl.BlockSpec((ROWS_BLK,), lambda i: (i,)),
            pl.BlockSpec((ROWS_BLK,), lambda i: (i,)),
            pl.BlockSpec((ROWS_BLK, D), lambda i: (i, 0)),
        ],
        out_specs=pl.BlockSpec((ROWS_BLK, D), lambda i: (i, 0)),
        out_shape=jax.ShapeDtypeStruct((N_NODES, D), jnp.float32),
    )(c0, c1, v)
    return out


# speedup vs baseline: 187.7159x; 1.0073x over previous
import functools

import jax
import jax.numpy as jnp
from jax import lax
from jax.experimental import pallas as pl
from jax.experimental.pallas import tpu as pltpu
from jax.experimental.pallas import tpu_sc as plsc

N_NODES = 10000
D = 128
CHUNK = 128
N_TILES = 16
N_CORES = 2
NW = N_CORES * N_TILES
N_PAD = 10240
SLICE = N_PAD // N_TILES


def _make_counts_kernel(main_rows: int, aux_rows: int):
    base_rows = (main_rows + aux_rows) // NW
    main_last = base_rows - aux_rows
    assert base_rows % 8 == 0 and main_last % 8 == 0 and main_last >= 0
    assert (NW - 1) * base_rows + main_last == main_rows
    mesh = plsc.VectorSubcoreMesh(core_axis_name="c", subcore_axis_name="s")

    group = 16
    assert base_rows % group == 0

    @functools.partial(
        pl.kernel,
        mesh=mesh,
        out_type=[jax.ShapeDtypeStruct((N_PAD,), jnp.float32),
                  jax.ShapeDtypeStruct((N_PAD,), jnp.float32)],
        scratch_types=[
            pltpu.VMEM((base_rows, CHUNK), jnp.int32),
            pltpu.VMEM((CHUNK,), jnp.float32),
            pltpu.VMEM((SLICE,), jnp.float32),
            pltpu.VMEM_SHARED((N_PAD,), jnp.float32),
            pltpu.SemaphoreType.DMA,
            pltpu.SemaphoreType.DMA,
        ],
    )
    def counts_kernel(idx_hbm, aux_hbm, out0_hbm, out1_hbm, idx_v, ones_v,
                      zero_v, counts_sh, stage_sem, scat_sem):
        cid = lax.axis_index("c")
        sid = lax.axis_index("s")
        wid = cid * N_TILES + sid
        is_last = wid == NW - 1

        @pl.when(jnp.logical_not(is_last))
        def _():
            pltpu.async_copy(idx_hbm.at[pl.ds(wid * base_rows, base_rows)],
                             idx_v, stage_sem)

        @pl.when(is_last)
        def _():
            pltpu.async_copy(
                idx_hbm.at[pl.ds((NW - 1) * base_rows, main_last)],
                idx_v.at[pl.ds(0, main_last)], stage_sem)
            pltpu.async_copy(aux_hbm,
                             idx_v.at[pl.ds(main_last, aux_rows)], stage_sem)

        for i in range(CHUNK // 16):
            ones_v[pl.ds(i * 16, 16)] = jnp.full((16,), 1.0, jnp.float32)
        for i in range(SLICE // 16):
            zero_v[pl.ds(i * 16, 16)] = jnp.zeros((16,), jnp.float32)

        pltpu.sync_copy(zero_v, counts_sh.at[pl.ds(sid * SLICE, SLICE)])
        pltpu.make_async_copy(
            idx_hbm.at[pl.ds(0, base_rows)], idx_v, stage_sem).wait()
        plsc.subcore_barrier()

        def body(g, carry):
            descs = [
                pltpu.async_copy(ones_v,
                                 counts_sh.at[idx_v.at[g * group + j]],
                                 scat_sem, add=True)
                for j in range(group)
            ]
            for d in descs:
                d.wait()
            return carry

        lax.fori_loop(0, base_rows // group, body, 0)

        plsc.subcore_barrier()
        @pl.when(cid == 0)
        def _():
            pltpu.sync_copy(counts_sh.at[pl.ds(sid * SLICE, SLICE)],
                            out0_hbm.at[pl.ds(sid * SLICE, SLICE)])

        @pl.when(cid == 1)
        def _():
            pltpu.sync_copy(counts_sh.at[pl.ds(sid * SLICE, SLICE)],
                            out1_hbm.at[pl.ds(sid * SLICE, SLICE)])

    return counts_kernel


ROWS_BLK = 2048


def _mask_mul_kernel(c0_ref, c1_ref, v_ref, out_ref):
    total = c0_ref[...] + c1_ref[...]
    m = (total > 0.0).astype(jnp.float32)
    m2 = jax.lax.broadcast_in_dim(m, (ROWS_BLK, D), (0,))
    out_ref[...] = v_ref[...] * m2


def kernel(q, k, v, self_indices, neighbor_indices):
    del q, k, neighbor_indices
    e = self_indices.shape[0]
    main_rows = e // CHUNK
    rows = -(-main_rows // (NW * 8)) * (NW * 8)
    pad_rows = rows - main_rows
    tail_rows = main_rows % 8
    aux_rows = pad_rows + tail_rows
    idx_flat = self_indices.astype(jnp.int32)
    idx2d = idx_flat.reshape(main_rows, CHUNK)
    pad = N_NODES + (jnp.arange(pad_rows * CHUNK, dtype=jnp.int32)
                     % (N_PAD - N_NODES))
    aux = jnp.concatenate(
        [idx_flat[(main_rows - tail_rows) * CHUNK:], pad]
    ).reshape(aux_rows, CHUNK)
    c0, c1 = _make_counts_kernel(main_rows - tail_rows, aux_rows)(idx2d, aux)
    out = pl.pallas_call(
        _mask_mul_kernel,
        grid=(-(-N_NODES // ROWS_BLK),),
        in_specs=[
            pl.BlockSpec((ROWS_BLK,), lambda i: (i,)),
            pl.BlockSpec((ROWS_BLK,), lambda i: (i,)),
            pl.BlockSpec((ROWS_BLK, D), lambda i: (i, 0)),
        ],
        out_specs=pl.BlockSpec((ROWS_BLK, D), lambda i: (i, 0)),
        out_shape=jax.ShapeDtypeStruct((N_NODES, D), jnp.float32),
    )(c0, c1, v)
    return out


# speedup vs baseline: 203.3674x; 1.0834x over previous
import functools

import jax
import jax.numpy as jnp
from jax import lax
from jax.experimental import pallas as pl
from jax.experimental.pallas import tpu as pltpu
from jax.experimental.pallas import tpu_sc as plsc

N_NODES = 10000
D = 128
CHUNK = 128
N_TILES = 16
N_CORES = 2
NW = N_CORES * N_TILES
N_PAD = 10240
SLICE = N_PAD // N_TILES


def _make_counts_kernel(main_rows: int, aux_rows: int):
    base_rows = (main_rows + aux_rows) // NW
    main_last = base_rows - aux_rows
    assert base_rows % 8 == 0 and main_last % 8 == 0 and main_last >= 0
    assert (NW - 1) * base_rows + main_last == main_rows
    mesh = plsc.VectorSubcoreMesh(core_axis_name="c", subcore_axis_name="s")

    group = 16
    assert base_rows % group == 0

    @functools.partial(
        pl.kernel,
        mesh=mesh,
        out_type=[jax.ShapeDtypeStruct((N_PAD,), jnp.float32),
                  jax.ShapeDtypeStruct((N_PAD,), jnp.float32)],
        scratch_types=[
            pltpu.VMEM((base_rows, CHUNK), jnp.int32),
            pltpu.VMEM((CHUNK,), jnp.float32),
            pltpu.VMEM((SLICE,), jnp.float32),
            pltpu.VMEM_SHARED((N_PAD,), jnp.float32),
            pltpu.SemaphoreType.DMA,
            pltpu.SemaphoreType.DMA,
        ],
    )
    def counts_kernel(idx_hbm, aux_hbm, out0_hbm, out1_hbm, idx_v, ones_v,
                      zero_v, counts_sh, stage_sem, scat_sem):
        cid = lax.axis_index("c")
        sid = lax.axis_index("s")
        wid = cid * N_TILES + sid
        is_last = wid == NW - 1

        @pl.when(jnp.logical_not(is_last))
        def _():
            pltpu.async_copy(idx_hbm.at[pl.ds(wid * base_rows, base_rows)],
                             idx_v, stage_sem)

        @pl.when(is_last)
        def _():
            pltpu.async_copy(
                idx_hbm.at[pl.ds((NW - 1) * base_rows, main_last)],
                idx_v.at[pl.ds(0, main_last)], stage_sem)
            pltpu.async_copy(aux_hbm,
                             idx_v.at[pl.ds(main_last, aux_rows)], stage_sem)

        for i in range(CHUNK // 16):
            ones_v[pl.ds(i * 16, 16)] = jnp.full((16,), 1.0, jnp.float32)
        for i in range(SLICE // 16):
            zero_v[pl.ds(i * 16, 16)] = jnp.zeros((16,), jnp.float32)

        pltpu.sync_copy(zero_v, counts_sh.at[pl.ds(sid * SLICE, SLICE)])
        pltpu.make_async_copy(
            idx_hbm.at[pl.ds(0, base_rows)], idx_v, stage_sem).wait()
        plsc.subcore_barrier()

        def body(g, carry):
            descs = [
                pltpu.async_copy(ones_v,
                                 counts_sh.at[idx_v.at[g * group + j]],
                                 scat_sem, add=True)
                for j in range(group)
            ]
            for d in descs:
                d.wait()
            return carry

        lax.fori_loop(0, base_rows // group, body, 0)

        plsc.subcore_barrier()
        @pl.when(cid == 0)
        def _():
            pltpu.sync_copy(counts_sh.at[pl.ds(sid * SLICE, SLICE)],
                            out0_hbm.at[pl.ds(sid * SLICE, SLICE)])

        @pl.when(cid == 1)
        def _():
            pltpu.sync_copy(counts_sh.at[pl.ds(sid * SLICE, SLICE)],
                            out1_hbm.at[pl.ds(sid * SLICE, SLICE)])

    return counts_kernel


ROWS_BLK = 5120


def _mask_mul_kernel(c0_ref, c1_ref, v_ref, out_ref):
    total = c0_ref[...] + c1_ref[...]
    m = (total > 0.0).astype(jnp.float32)
    m2 = jax.lax.broadcast_in_dim(m, (ROWS_BLK, D), (0,))
    out_ref[...] = v_ref[...] * m2


def kernel(q, k, v, self_indices, neighbor_indices):
    del q, k, neighbor_indices
    e = self_indices.shape[0]
    main_rows = e // CHUNK
    rows = -(-main_rows // (NW * 8)) * (NW * 8)
    pad_rows = rows - main_rows
    tail_rows = main_rows % 8
    aux_rows = pad_rows + tail_rows
    idx_flat = self_indices.astype(jnp.int32)
    idx2d = idx_flat.reshape(main_rows, CHUNK)
    pad = N_NODES + (jnp.arange(pad_rows * CHUNK, dtype=jnp.int32)
                     % (N_PAD - N_NODES))
    aux = jnp.concatenate(
        [idx_flat[(main_rows - tail_rows) * CHUNK:], pad]
    ).reshape(aux_rows, CHUNK)
    c0, c1 = _make_counts_kernel(main_rows - tail_rows, aux_rows)(idx2d, aux)
    out = pl.pallas_call(
        _mask_mul_kernel,
        grid=(-(-N_NODES // ROWS_BLK),),
        in_specs=[
            pl.BlockSpec((ROWS_BLK,), lambda i: (i,)),
            pl.BlockSpec((ROWS_BLK,), lambda i: (i,)),
            pl.BlockSpec((ROWS_BLK, D), lambda i: (i, 0)),
        ],
        out_specs=pl.BlockSpec((ROWS_BLK, D), lambda i: (i, 0)),
        out_shape=jax.ShapeDtypeStruct((N_NODES, D), jnp.float32),
    )(c0, c1, v)
    return out
